# baseline (speedup 1.0000x reference)
import functools

import jax
import jax.numpy as jnp
from jax import lax
from jax.experimental import pallas as pl
from jax.experimental.pallas import tpu as pltpu
from jax.experimental.pallas import tpu_sc as plsc

D = 64
N = 100000
B = 16384
BH = B // 2
NC = 2
NS = 16
NW = NC * NS
PER_W = B // NW
CH = 128
_SUP = 2 * CH
_NSUP = PER_W // _SUP

_MESH = plsc.VectorSubcoreMesh(
    core_axis_name="c", subcore_axis_name="s", num_cores=NC, num_subcores=NS
)

_G_ET = (
    (0, ((0, 0),)),
    (1, ((1, 0),)),
    (2, ((2, 0),)),
    (3, ((3, 0),)),
    (4, ((4, 1),)),
    (5, ((5, 1),)),
)
_G_BC = (
    (0, ((0, 0), (1, 1))),
    (1, ((2, 0), (3, 1))),
)
_G_HR = (
    (0, ((0, 0),)),
    (1, ((1, 0),)),
    (2, ((2, 1),)),
    (3, ((3, 1),)),
)


_EC = 12288
_NEB = (N + _EC - 1) // _EC
_PAIR = jax.ShapeDtypeStruct((N, 2 * D), jnp.float32)
_STAT = jax.ShapeDtypeStruct((N,), jnp.float32)
_IN_SPEC = pl.BlockSpec((D, _EC), lambda i: (0, i))
_PAIR_SPEC = pl.BlockSpec((_EC, 2 * D), lambda i: (i, 0))
_STAT_SPEC = pl.BlockSpec((_EC,), lambda i: (i,))
_ARB = pltpu.CompilerParams(dimension_semantics=("arbitrary",))


def _repack_et_body(aT, bT, p, se):
    a = aT[...]
    p[:, :D] = a.T
    p[:, D:] = bT[...].T
    t = jnp.sum(a * a, axis=0) - jnp.float32(1.0)
    se[...] = t * t


def _repack_bc_body(aT, bT, p, sa, sc):
    one = jnp.float32(1.0)
    a = aT[...]
    b = bT[...]
    p[:, :D] = a.T
    p[:, D:] = b.T
    ta = a * (one - a)
    tb = b * (one - b)
    sa[...] = jnp.sum(ta * ta, axis=0) + jnp.sum(tb * tb, axis=0)
    sc[...] = (jnp.maximum(one - jnp.sum(jnp.abs(a), axis=0), 0.0)
               + jnp.maximum(one - jnp.sum(jnp.abs(b), axis=0), 0.0))


def _repack_hr_body(aT, bT, p):
    p[:, :D] = aT[...].T
    p[:, D:] = bT[...].T


def _repack_et(aT, bT):
    return pl.pallas_call(
        _repack_et_body, grid=(_NEB,), in_specs=[_IN_SPEC] * 2,
        out_specs=[_PAIR_SPEC, _STAT_SPEC], out_shape=[_PAIR, _STAT],
        compiler_params=_ARB)(aT, bT)


def _repack_bc(aT, bT):
    return pl.pallas_call(
        _repack_bc_body, grid=(_NEB,), in_specs=[_IN_SPEC] * 2,
        out_specs=[_PAIR_SPEC, _STAT_SPEC, _STAT_SPEC],
        out_shape=[_PAIR, _STAT, _STAT],
        compiler_params=_ARB)(aT, bT)


def _repack_hr(aT, bT):
    return pl.pallas_call(
        _repack_hr_body, grid=(_NEB,), in_specs=[_IN_SPEC] * 2,
        out_specs=_PAIR_SPEC, out_shape=_PAIR,
        compiler_params=_ARB)(aT, bT)


def _make_sc_gather(gathers, n_idx, n_out, stat_specs):
    n_stats = len(stat_specs)
    out_types = [jax.ShapeDtypeStruct((n_out, BH, 2 * D), jnp.float32)]
    out_types += [jax.ShapeDtypeStruct((B,), jnp.float32)] * n_stats

    @functools.partial(
        pl.kernel,
        out_type=tuple(out_types),
        mesh=_MESH,
        scratch_types=[
            pltpu.VMEM((n_idx, _NSUP, _SUP), jnp.int32),
            pltpu.VMEM((3, _SUP, 2 * D), jnp.float32),
            pltpu.VMEM((_SUP,), jnp.float32),
            pltpu.SemaphoreType.DMA,
            pltpu.SemaphoreType.DMA,
            pltpu.SemaphoreType.DMA,
            pltpu.SemaphoreType.DMA,
            pltpu.SemaphoreType.DMA,
            pltpu.SemaphoreType.DMA,
            pltpu.SemaphoreType.DMA,
        ],
        compiler_params=pltpu.CompilerParams(use_tc_tiling_on_sc=False),
    )
    def sc_gather(*args):
        tbl = args[0]
        stats = args[1:1 + n_stats]
        idx_hbm = args[1 + n_stats]
        out = args[2 + n_stats]
        souts = args[3 + n_stats:3 + 2 * n_stats]
        idx_v, rows, sbuf, sem_i, sg0, sg1, sg2, so0, so1, so2 = \
            args[3 + 2 * n_stats:]
        wid = lax.axis_index("s") * NC + lax.axis_index("c")
        base = wid * PER_W
        half = base // BH
        row0 = base - half * BH
        sg = (sg0, sg1, sg2)
        so = (so0, so1, so2)
        pltpu.async_copy(idx_hbm.at[:, wid], idx_v, sem_i).wait()

        items = [(ii, outs, s) for ii, outs in gathers
                 for s in range(_NSUP)]
        K = len(items)
        g_cp = [None] * K
        o_cp = [None] * K

        def start_gather(k):
            ii, _, s = items[k]
            b = k % 3
            g_cp[k] = pltpu.async_copy(
                tbl.at[idx_v.at[ii, s]], rows.at[b], sg[b])

        def start_copyouts(k):
            ii, outs, s = items[k]
            b = k % 3
            cs = []
            for g, h in outs:
                cs.append(pltpu.async_copy(
                    rows.at[b, :, pl.ds(h * D, D)],
                    out.at[g, pl.ds(row0 + s * _SUP, _SUP),
                           pl.ds(half * D, D)],
                    so[b]))
            o_cp[k] = cs

        start_gather(0)
        start_gather(1)
        for k in range(K):
            if k + 2 < K:
                if k >= 1:
                    for c in o_cp[k - 1]:
                        c.wait()
                start_gather(k + 2)
            g_cp[k].wait()
            start_copyouts(k)
        for k in range(max(0, K - 3), K):
            for c in o_cp[k]:
                c.wait()

        for (ii, sp), sout in zip(stat_specs, souts):
            for s in range(_NSUP):
                pltpu.sync_copy(stats[sp].at[idx_v.at[ii, s]], sbuf)
                pltpu.sync_copy(sbuf,
                                sout.at[pl.ds(base + s * _SUP, _SUP)])

    return sc_gather


_sc_et = _make_sc_gather(_G_ET, 7, 6, ((6, 0),))
_sc_bc = _make_sc_gather(_G_BC, 3, 4, ((2, 0), (2, 1)))
_sc_hr = _make_sc_gather(_G_HR, 4, 4, ())


_RH = 512


def _half_mask():
    row = lax.broadcasted_iota(jnp.int32, (2, 2 * D), 0)
    lane = lax.broadcasted_iota(jnp.int32, (2, 2 * D), 1)
    return jnp.where((lane < D) == (row == 0), jnp.float32(1), jnp.float32(0))


def _halfsum(v, mask):
    return lax.dot_general(
        mask, v, (((1,), (1,)), ((), ())),
        preferred_element_type=jnp.float32,
        precision=lax.Precision.HIGHEST)


def _tc_batch_body(m_ref, et_ref, bc_ref, hr_ref, o1, o2, o6):
    m = m_ref[0, 0]
    one = jnp.float32(1.0)
    et = et_ref[...]
    bc = bc_ref[...]
    hr = hr_ref[...]
    aBHEE, aBTEE, nABHEE, nABTEE = et[0], et[1], et[2], et[3]
    aTailE, nTailE = et[4], et[5]
    aBCHE, aBCTE, nABCHE, nABCTE = bc[0], bc[1], bc[2], bc[3]
    aHeadE, nHeadE, aRelE, nRelE = hr[0], hr[1], hr[2], hr[3]

    msk = _half_mask()
    tmpBE2CH = (one - aBCHE) * aBHEE
    tmpBE2CT = (one - aBCTE) * aBTEE
    tE = _halfsum(jnp.abs(aHeadE + aRelE - aTailE), msk)
    s1 = _halfsum(tmpBE2CH * tmpBE2CH + tmpBE2CT * tmpBE2CT, msk)
    tmpNBE2CH = (one - nABCHE) * nABHEE
    tmpNBE2CT = (one - nABCTE) * nABTEE
    ntE = _halfsum(jnp.abs(nHeadE + nRelE - nTailE), msk)
    s2 = _halfsum(tmpNBE2CH * tmpNBE2CH + tmpNBE2CT * tmpNBE2CT, msk)
    o1[...] = s1 + tE
    o2[...] = jnp.maximum(m - (s2 + ntE), 0.0)
    o6[...] = jnp.maximum(m + tE - ntE, 0.0)


def _tc_batch(margin2d, et, bc, hr):
    blk = lambda n: pl.BlockSpec((n, _RH, 2 * D), lambda i: (0, i, 0))
    return pl.pallas_call(
        _tc_batch_body,
        grid=(BH // _RH,),
        in_specs=[pl.BlockSpec((1, 1), lambda i: (0, 0)),
                  blk(6), blk(4), blk(4)],
        out_specs=[pl.BlockSpec((2, _RH), lambda i: (0, i))] * 3,
        out_shape=[jax.ShapeDtypeStruct((2, BH), jnp.float32)] * 3,
        compiler_params=_ARB,
    )(margin2d, et, bc, hr)


def kernel(aBHE, aBTE, aBC, aHead, aTail, aRelation, nABHE, nABTE, nABC,
           nHead, nTail, nRelation, uniqE, uniqBC, lossMargin, device,
           entityEmbed, bConceptHEmbed, bConceptTEmbed, headEmbed,
           tailEmbed, relationEmbed):
    i32 = lambda a: a.astype(jnp.int32)
    shp = lambda x: x.reshape(x.shape[0], NW, _NSUP, _SUP)
    idx_et = shp(jnp.stack([i32(aBHE), i32(aBTE), i32(nABHE), i32(nABTE),
                            i32(aTail), i32(nTail), i32(uniqE)]))
    idx_bc = shp(jnp.stack([i32(aBC), i32(nABC), i32(uniqBC)]))
    idx_hr = shp(jnp.stack([i32(aHead), i32(nHead), i32(aRelation),
                            i32(nRelation)]))
    sw = lambda t: jnp.swapaxes(t, 0, 1)
    p_et, statE = _repack_et(sw(entityEmbed), sw(tailEmbed))
    g_et, o3 = _sc_et(p_et, statE, idx_et)
    p_bc, statA, statC = _repack_bc(sw(bConceptHEmbed), sw(bConceptTEmbed))
    g_bc, o4, o5 = _sc_bc(p_bc, statA, statC, idx_bc)
    p_hr = _repack_hr(sw(headEmbed), sw(relationEmbed))
    (g_hr,) = _sc_hr(p_hr, idx_hr)
    margin2d = jnp.asarray(lossMargin, jnp.float32).reshape(1, 1)
    o1, o2, o6 = _tc_batch(margin2d, g_et, g_bc, g_hr)
    join = lambda t: jnp.concatenate([t[0], t[1]])
    return (join(o1), join(o2), o3, o4, o5, join(o6))


# speedup vs baseline: 1.0065x; 1.0065x over previous
import functools

import jax
import jax.numpy as jnp
from jax import lax
from jax.experimental import pallas as pl
from jax.experimental.pallas import tpu as pltpu
from jax.experimental.pallas import tpu_sc as plsc

D = 64
N = 100000
B = 16384
BH = B // 2
NC = 2
NS = 16
NW = NC * NS
PER_W = B // NW
CH = 128
_SUP = 2 * CH
_NSUP = PER_W // _SUP

_MESH = plsc.VectorSubcoreMesh(
    core_axis_name="c", subcore_axis_name="s", num_cores=NC, num_subcores=NS
)

_G_ET = (
    (0, ((0, 0),)),
    (1, ((1, 0),)),
    (2, ((2, 0),)),
    (3, ((3, 0),)),
    (4, ((4, 1),)),
    (5, ((5, 1),)),
)
_G_BC = (
    (0, ((0, 0), (1, 1))),
    (1, ((2, 0), (3, 1))),
)
_G_HR = (
    (0, ((0, 0),)),
    (1, ((1, 0),)),
    (2, ((2, 1),)),
    (3, ((3, 1),)),
)


_EC = 8192
_NEB = (N + _EC - 1) // _EC
_PAIR = jax.ShapeDtypeStruct((N, 2 * D), jnp.float32)
_STAT = jax.ShapeDtypeStruct((N,), jnp.float32)
_IN_SPEC = pl.BlockSpec((D, _EC), lambda i: (0, i))
_PAIR_SPEC = pl.BlockSpec((_EC, 2 * D), lambda i: (i, 0))
_STAT_SPEC = pl.BlockSpec((_EC,), lambda i: (i,))
_ARB = pltpu.CompilerParams(dimension_semantics=("arbitrary",))


def _repack_et_body(aT, bT, p, se):
    a = aT[...]
    p[:, :D] = a.T
    p[:, D:] = bT[...].T
    t = jnp.sum(a * a, axis=0) - jnp.float32(1.0)
    se[...] = t * t


def _repack_bc_body(aT, bT, p, sa, sc):
    one = jnp.float32(1.0)
    a = aT[...]
    b = bT[...]
    p[:, :D] = a.T
    p[:, D:] = b.T
    ta = a * (one - a)
    tb = b * (one - b)
    sa[...] = jnp.sum(ta * ta, axis=0) + jnp.sum(tb * tb, axis=0)
    sc[...] = (jnp.maximum(one - jnp.sum(jnp.abs(a), axis=0), 0.0)
               + jnp.maximum(one - jnp.sum(jnp.abs(b), axis=0), 0.0))


def _repack_hr_body(aT, bT, p):
    p[:, :D] = aT[...].T
    p[:, D:] = bT[...].T


def _repack_et(aT, bT):
    return pl.pallas_call(
        _repack_et_body, grid=(_NEB,), in_specs=[_IN_SPEC] * 2,
        out_specs=[_PAIR_SPEC, _STAT_SPEC], out_shape=[_PAIR, _STAT],
        compiler_params=_ARB)(aT, bT)


def _repack_bc(aT, bT):
    return pl.pallas_call(
        _repack_bc_body, grid=(_NEB,), in_specs=[_IN_SPEC] * 2,
        out_specs=[_PAIR_SPEC, _STAT_SPEC, _STAT_SPEC],
        out_shape=[_PAIR, _STAT, _STAT],
        compiler_params=_ARB)(aT, bT)


def _repack_hr(aT, bT):
    return pl.pallas_call(
        _repack_hr_body, grid=(_NEB,), in_specs=[_IN_SPEC] * 2,
        out_specs=_PAIR_SPEC, out_shape=_PAIR,
        compiler_params=_ARB)(aT, bT)


def _make_sc_gather(gathers, n_idx, n_out, stat_specs):
    n_stats = len(stat_specs)
    out_types = [jax.ShapeDtypeStruct((n_out, BH, 2 * D), jnp.float32)]
    out_types += [jax.ShapeDtypeStruct((B,), jnp.float32)] * n_stats

    @functools.partial(
        pl.kernel,
        out_type=tuple(out_types),
        mesh=_MESH,
        scratch_types=[
            pltpu.VMEM((n_idx, _NSUP, _SUP), jnp.int32),
            pltpu.VMEM((3, _SUP, 2 * D), jnp.float32),
            pltpu.VMEM((_SUP,), jnp.float32),
            pltpu.SemaphoreType.DMA,
            pltpu.SemaphoreType.DMA,
            pltpu.SemaphoreType.DMA,
            pltpu.SemaphoreType.DMA,
            pltpu.SemaphoreType.DMA,
            pltpu.SemaphoreType.DMA,
            pltpu.SemaphoreType.DMA,
        ],
        compiler_params=pltpu.CompilerParams(use_tc_tiling_on_sc=False),
    )
    def sc_gather(*args):
        tbl = args[0]
        stats = args[1:1 + n_stats]
        idx_hbm = args[1 + n_stats]
        out = args[2 + n_stats]
        souts = args[3 + n_stats:3 + 2 * n_stats]
        idx_v, rows, sbuf, sem_i, sg0, sg1, sg2, so0, so1, so2 = \
            args[3 + 2 * n_stats:]
        wid = lax.axis_index("s") * NC + lax.axis_index("c")
        base = wid * PER_W
        half = base // BH
        row0 = base - half * BH
        sg = (sg0, sg1, sg2)
        so = (so0, so1, so2)
        pltpu.async_copy(idx_hbm.at[:, wid], idx_v, sem_i).wait()

        items = [(ii, outs, s) for ii, outs in gathers
                 for s in range(_NSUP)]
        K = len(items)
        g_cp = [None] * K
        o_cp = [None] * K

        def start_gather(k):
            ii, _, s = items[k]
            b = k % 3
            g_cp[k] = pltpu.async_copy(
                tbl.at[idx_v.at[ii, s]], rows.at[b], sg[b])

        def start_copyouts(k):
            ii, outs, s = items[k]
            b = k % 3
            cs = []
            for g, h in outs:
                cs.append(pltpu.async_copy(
                    rows.at[b, :, pl.ds(h * D, D)],
                    out.at[g, pl.ds(row0 + s * _SUP, _SUP),
                           pl.ds(half * D, D)],
                    so[b]))
            o_cp[k] = cs

        start_gather(0)
        start_gather(1)
        for k in range(K):
            if k + 2 < K:
                if k >= 1:
                    for c in o_cp[k - 1]:
                        c.wait()
                start_gather(k + 2)
            g_cp[k].wait()
            start_copyouts(k)
        for k in range(max(0, K - 3), K):
            for c in o_cp[k]:
                c.wait()

        for (ii, sp), sout in zip(stat_specs, souts):
            for s in range(_NSUP):
                pltpu.sync_copy(stats[sp].at[idx_v.at[ii, s]], sbuf)
                pltpu.sync_copy(sbuf,
                                sout.at[pl.ds(base + s * _SUP, _SUP)])

    return sc_gather


_sc_et = _make_sc_gather(_G_ET, 7, 6, ((6, 0),))
_sc_bc = _make_sc_gather(_G_BC, 3, 4, ((2, 0), (2, 1)))
_sc_hr = _make_sc_gather(_G_HR, 4, 4, ())


_RH = 512


def _half_mask():
    row = lax.broadcasted_iota(jnp.int32, (2, 2 * D), 0)
    lane = lax.broadcasted_iota(jnp.int32, (2, 2 * D), 1)
    return jnp.where((lane < D) == (row == 0), jnp.float32(1), jnp.float32(0))


def _halfsum(v, mask):
    return lax.dot_general(
        mask, v, (((1,), (1,)), ((), ())),
        preferred_element_type=jnp.float32,
        precision=lax.Precision.HIGHEST)


def _tc_batch_body(m_ref, et_ref, bc_ref, hr_ref, o1, o2, o6):
    m = m_ref[0, 0]
    one = jnp.float32(1.0)
    et = et_ref[...]
    bc = bc_ref[...]
    hr = hr_ref[...]
    aBHEE, aBTEE, nABHEE, nABTEE = et[0], et[1], et[2], et[3]
    aTailE, nTailE = et[4], et[5]
    aBCHE, aBCTE, nABCHE, nABCTE = bc[0], bc[1], bc[2], bc[3]
    aHeadE, nHeadE, aRelE, nRelE = hr[0], hr[1], hr[2], hr[3]

    msk = _half_mask()
    tmpBE2CH = (one - aBCHE) * aBHEE
    tmpBE2CT = (one - aBCTE) * aBTEE
    tE = _halfsum(jnp.abs(aHeadE + aRelE - aTailE), msk)
    s1 = _halfsum(tmpBE2CH * tmpBE2CH + tmpBE2CT * tmpBE2CT, msk)
    tmpNBE2CH = (one - nABCHE) * nABHEE
    tmpNBE2CT = (one - nABCTE) * nABTEE
    ntE = _halfsum(jnp.abs(nHeadE + nRelE - nTailE), msk)
    s2 = _halfsum(tmpNBE2CH * tmpNBE2CH + tmpNBE2CT * tmpNBE2CT, msk)
    o1[...] = s1 + tE
    o2[...] = jnp.maximum(m - (s2 + ntE), 0.0)
    o6[...] = jnp.maximum(m + tE - ntE, 0.0)


def _tc_batch(margin2d, et, bc, hr):
    blk = lambda n: pl.BlockSpec((n, _RH, 2 * D), lambda i: (0, i, 0))
    return pl.pallas_call(
        _tc_batch_body,
        grid=(BH // _RH,),
        in_specs=[pl.BlockSpec((1, 1), lambda i: (0, 0)),
                  blk(6), blk(4), blk(4)],
        out_specs=[pl.BlockSpec((2, _RH), lambda i: (0, i))] * 3,
        out_shape=[jax.ShapeDtypeStruct((2, BH), jnp.float32)] * 3,
        compiler_params=_ARB,
    )(margin2d, et, bc, hr)


def kernel(aBHE, aBTE, aBC, aHead, aTail, aRelation, nABHE, nABTE, nABC,
           nHead, nTail, nRelation, uniqE, uniqBC, lossMargin, device,
           entityEmbed, bConceptHEmbed, bConceptTEmbed, headEmbed,
           tailEmbed, relationEmbed):
    i32 = lambda a: a.astype(jnp.int32)
    shp = lambda x: x.reshape(x.shape[0], NW, _NSUP, _SUP)
    idx_et = shp(jnp.stack([i32(aBHE), i32(aBTE), i32(nABHE), i32(nABTE),
                            i32(aTail), i32(nTail), i32(uniqE)]))
    idx_bc = shp(jnp.stack([i32(aBC), i32(nABC), i32(uniqBC)]))
    idx_hr = shp(jnp.stack([i32(aHead), i32(nHead), i32(aRelation),
                            i32(nRelation)]))
    sw = lambda t: jnp.swapaxes(t, 0, 1)
    p_et, statE = _repack_et(sw(entityEmbed), sw(tailEmbed))
    g_et, o3 = _sc_et(p_et, statE, idx_et)
    p_bc, statA, statC = _repack_bc(sw(bConceptHEmbed), sw(bConceptTEmbed))
    g_bc, o4, o5 = _sc_bc(p_bc, statA, statC, idx_bc)
    p_hr = _repack_hr(sw(headEmbed), sw(relationEmbed))
    (g_hr,) = _sc_hr(p_hr, idx_hr)
    margin2d = jnp.asarray(lossMargin, jnp.float32).reshape(1, 1)
    o1, o2, o6 = _tc_batch(margin2d, g_et, g_bc, g_hr)
    join = lambda t: jnp.concatenate([t[0], t[1]])
    return (join(o1), join(o2), o3, o4, o5, join(o6))
